# Initial kernel scaffold; baseline (speedup 1.0000x reference)
import jax
import jax.numpy as jnp
from jax.experimental import pallas as pl


def kernel(positions, weight):
    raise NotImplementedError("write your pallas kernel here")


# speedup vs baseline: 2.3729x; 2.3729x over previous
import functools

import jax
import jax.numpy as jnp
from jax import lax
from jax.experimental import pallas as pl
from jax.experimental.pallas import tpu as pltpu
from jax.experimental.pallas import tpu_sc as plsc


def _make_sc_gather(B, D, n_rows):
    info = plsc.get_sparse_core_info()
    NC, NS = info.num_cores, info.num_subcores
    NW = NC * NS
    assert B % NW == 0
    b_per_w = B // NW
    C = 32
    NBUF = 2
    assert b_per_w % (C * NBUF) == 0
    n_chunks = b_per_w // C

    mesh = plsc.VectorSubcoreMesh(core_axis_name="c", subcore_axis_name="s")

    @functools.partial(
        pl.kernel,
        mesh=mesh,
        out_type=jax.ShapeDtypeStruct((B, D), jnp.float32),
        scratch_types=[
            pltpu.VMEM((n_chunks, C), jnp.int32),
            pltpu.VMEM((NBUF, C, D), jnp.float32),
            pltpu.SemaphoreType.DMA((NBUF,)),
        ],
    )
    def gather_kernel(idx_hbm, table_hbm, out_hbm, idx_v, rows_v, gsem):
        wid = lax.axis_index("s") * NC + lax.axis_index("c")
        base = wid * b_per_w
        pltpu.sync_copy(idx_hbm.at[wid], idx_v)

        for b in range(NBUF):
            pltpu.async_copy(table_hbm.at[idx_v.at[b]], rows_v.at[b],
                             gsem.at[b])

        def body(g, carry):
            for b in range(NBUF):
                c = g * NBUF + b
                pltpu.make_async_copy(table_hbm.at[idx_v.at[c]],
                                      rows_v.at[b], gsem.at[b]).wait()
                pltpu.sync_copy(rows_v.at[b],
                                out_hbm.at[pl.ds(base + c * C, C)])
                nxt = c + NBUF

                @pl.when(nxt < n_chunks)
                def _():
                    pltpu.async_copy(table_hbm.at[idx_v.at[nxt]],
                                     rows_v.at[b], gsem.at[b])
            return carry

        lax.fori_loop(0, n_chunks // NBUF, body, 0)

    return gather_kernel


@jax.jit
def kernel(positions, weight):
    n_rows, d = weight.shape
    bsz, seq = positions.shape
    B = bsz * seq
    info = plsc.get_sparse_core_info()
    NW = info.num_cores * info.num_subcores
    C = 32
    idx = positions.reshape(NW, B // (NW * C), C).astype(jnp.int32)
    out = _make_sc_gather(B, d, n_rows)(idx, weight)
    return out.reshape(bsz, seq, d)


# speedup vs baseline: 2.3854x; 1.0052x over previous
import functools

import jax
import jax.numpy as jnp
from jax import lax
from jax.experimental import pallas as pl
from jax.experimental.pallas import tpu as pltpu
from jax.experimental.pallas import tpu_sc as plsc


_CHUNK = 16
_NBUF = 4


def _make_sc_gather(B, D, n_rows):
    info = plsc.get_sparse_core_info()
    NC, NS = info.num_cores, info.num_subcores
    NW = NC * NS
    assert B % NW == 0
    b_per_w = B // NW
    C = _CHUNK
    NBUF = _NBUF
    assert b_per_w % (C * NBUF) == 0
    n_chunks = b_per_w // C

    mesh = plsc.VectorSubcoreMesh(core_axis_name="c", subcore_axis_name="s")

    @functools.partial(
        pl.kernel,
        mesh=mesh,
        out_type=jax.ShapeDtypeStruct((B, D), jnp.float32),
        scratch_types=[
            pltpu.VMEM((n_chunks, C), jnp.int32),
            pltpu.VMEM((NBUF, C, D), jnp.float32),
            pltpu.SemaphoreType.DMA((NBUF,)),
            pltpu.SemaphoreType.DMA((NBUF,)),
        ],
    )
    def gather_kernel(idx_hbm, table_hbm, out_hbm, idx_v, rows_v, gsem, wsem):
        wid = lax.axis_index("s") * NC + lax.axis_index("c")
        base = wid * b_per_w
        pltpu.sync_copy(idx_hbm.at[wid], idx_v)

        def gather_desc(c, b):
            return pltpu.make_async_copy(table_hbm.at[idx_v.at[c]],
                                         rows_v.at[b], gsem.at[b])

        def wb_desc(c, b):
            return pltpu.make_async_copy(rows_v.at[b],
                                         out_hbm.at[pl.ds(base + c * C, C)],
                                         wsem.at[b])

        for b in range(NBUF - 1):
            gather_desc(b, b).start()

        def body(g, carry):
            for b in range(NBUF):
                c = g * NBUF + b
                gather_desc(c, b).wait()
                wb_desc(c, b).start()
                nxt = c + NBUF - 1
                nb = (b + NBUF - 1) % NBUF

                @pl.when(nxt < n_chunks)
                def _():
                    @pl.when(c >= 1)
                    def _():
                        wb_desc(c - 1, nb).wait()

                    gather_desc(nxt, nb).start()
            return carry

        lax.fori_loop(0, n_chunks // NBUF, body, 0)

        for j in range(NBUF):
            c = n_chunks - NBUF + j
            wb_desc(c, c % NBUF).wait()

    return gather_kernel


@jax.jit
def kernel(positions, weight):
    n_rows, d = weight.shape
    bsz, seq = positions.shape
    B = bsz * seq
    info = plsc.get_sparse_core_info()
    NW = info.num_cores * info.num_subcores
    C = _CHUNK
    idx = positions.reshape(NW, B // (NW * C), C).astype(jnp.int32)
    out = _make_sc_gather(B, d, n_rows)(idx, weight)
    return out.reshape(bsz, seq, d)
